# baseline (speedup 1.0000x reference)
import functools

import jax
import jax.numpy as jnp
from jax import lax
from jax.experimental import pallas as pl
from jax.experimental.pallas import tpu as pltpu
from jax.experimental.pallas import tpu_sc as plsc

N = 10000
E = 320000
D_IN = 32
D = 128
D_EDGE = 16
G = 64

NC = 2
NS = 16
NW = NC * NS
L = 16

EPW = E // NW
CH = 40
NCHUNK = EPW // CH
ZR = 200
NZCH = N // ZR
ZT = (NZCH + NS - 1) // NS

_f32 = jnp.float32


@functools.lru_cache(maxsize=None)
def _sc_mesh():
    return plsc.VectorSubcoreMesh(
        core_axis_name="c", subcore_axis_name="s",
        num_cores=NC, num_subcores=NS)


NBA = 2000


TW = 2 * D


def _nodeproj_body(x_ref, cpad_ref, wemb_ref, bemb_ref, wa_ref, wb_ref,
                   h_ref, ta_ref, tb_ref):
    h = jnp.dot(x_ref[...], wemb_ref[...],
                preferred_element_type=_f32) + bemb_ref[...]
    h_ref[...] = h
    cp = cpad_ref[...]
    ha = jnp.dot(h, wa_ref[...], preferred_element_type=_f32)
    hb = jnp.dot(h, wb_ref[...], preferred_element_type=_f32)
    ta_ref[...] = jnp.concatenate([ha, cp], axis=1)
    tb_ref[...] = jnp.concatenate([hb, cp], axis=1)


def _nodeproj(x, cpad, wemb, bemb, wa, wb):
    full = lambda s: pl.BlockSpec(s, lambda i: (0, 0))
    return pl.pallas_call(
        _nodeproj_body,
        grid=(N // NBA,),
        in_specs=[
            pl.BlockSpec((NBA, D_IN), lambda i: (i, 0)),
            pl.BlockSpec((NBA, D), lambda i: (i, 0)),
            full((D_IN, D)), full((1, D)), full((D, D)), full((D, D)),
        ],
        out_specs=[
            pl.BlockSpec((NBA, D), lambda i: (i, 0)),
            pl.BlockSpec((NBA, TW), lambda i: (i, 0)),
            pl.BlockSpec((NBA, TW), lambda i: (i, 0)),
        ],
        out_shape=[
            jax.ShapeDtypeStruct((N, D), _f32),
            jax.ShapeDtypeStruct((N, TW), _f32),
            jax.ShapeDtypeStruct((N, TW), _f32),
        ],
    )(x, cpad, wemb, bemb, wa, wb)


CW = 16
_u8 = jnp.uint8


def _gather_body(rowr_hbm, colr_hbm, taba_hbm, tabb_hbm,
                 u_hbm, rel_hbm,
                 idxr, idxc, bufa0, bufb0, bufa1, bufb1, ubuf, rbuf,
                 sema0, semb0, sema1, semb1, semw):
    c = lax.axis_index("c")
    s = lax.axis_index("s")
    w = s * NC + c
    e_base = w * EPW

    pltpu.sync_copy(rowr_hbm.at[w], idxr)
    pltpu.sync_copy(colr_hbm.at[w], idxc)

    bufa = (bufa0, bufa1)
    bufb = (bufb0, bufb1)
    sema = (sema0, sema1)
    semb = (semb0, semb1)

    def fire(j, st):
        pltpu.async_copy(taba_hbm.at[idxr.at[j]], bufa[st], sema[st])
        pltpu.async_copy(tabb_hbm.at[idxc.at[j]], bufb[st], semb[st])

    def wait_g(st):
        pltpu.make_async_copy(taba_hbm.at[idxr.at[0]], bufa[st],
                              sema[st]).wait()
        pltpu.make_async_copy(tabb_hbm.at[idxc.at[0]], bufb[st],
                              semb[st]).wait()

    def compute(st):
        a = bufa[st]
        b = bufb[st]

        def crow(r, carry):
            for k in range(D // L):
                ubuf[r, pl.ds(k * L, L)] = (a[r, pl.ds(k * L, L)]
                                            + b[r, pl.ds(k * L, L)])
            rbuf[pl.ds(r * L, L)] = (a[r, pl.ds(D, L)]
                                     - b[r, pl.ds(D, L)])
            return carry

        lax.fori_loop(0, CH, crow, 0)

    def fire_w(j):
        e0 = e_base + j * CH
        pltpu.async_copy(ubuf, u_hbm.at[pl.ds(e0, CH)], semw)
        pltpu.async_copy(rbuf, rel_hbm.at[pl.ds(e0 * CW, CH * CW)], semw)

    def wait_w(j):
        e0 = e_base + j * CH
        pltpu.make_async_copy(ubuf, u_hbm.at[pl.ds(e0, CH)], semw).wait()
        pltpu.make_async_copy(rbuf, rel_hbm.at[pl.ds(e0 * CW, CH * CW)],
                              semw).wait()

    fire(0, 0)
    wait_g(0)
    fire(1, 1)
    compute(0)
    fire_w(0)

    last = NCHUNK - 1

    def body(i, carry):
        j = 2 * i + 1
        wait_g(1)
        fire(j + 1, 0)
        wait_w(j - 1)
        compute(1)
        fire_w(j)
        j2 = j + 1
        wait_g(0)
        fire(jnp.minimum(j2 + 1, last), 1)
        wait_w(j2 - 1)
        compute(0)
        fire_w(j2)
        return carry

    lax.fori_loop(0, (NCHUNK - 2) // 2, body, 0)
    wait_g(1)
    wait_w(last - 1)
    compute(1)
    fire_w(last)
    wait_w(last)


@functools.lru_cache(maxsize=None)
def _gather_kernel():
    return pl.kernel(
        _gather_body,
        out_type=[
            jax.ShapeDtypeStruct((E, D), _f32),
            jax.ShapeDtypeStruct((E * CW,), _f32),
        ],
        mesh=_sc_mesh(),
        scratch_types=[
            pltpu.VMEM((NCHUNK, CH), jnp.int32),
            pltpu.VMEM((NCHUNK, CH), jnp.int32),
            pltpu.VMEM((CH, TW), _f32),
            pltpu.VMEM((CH, TW), _f32),
            pltpu.VMEM((CH, TW), _f32),
            pltpu.VMEM((CH, TW), _f32),
            pltpu.VMEM((CH, D), _f32),
            pltpu.VMEM((CH * CW,), _f32),
            pltpu.SemaphoreType.DMA,
            pltpu.SemaphoreType.DMA,
            pltpu.SemaphoreType.DMA,
            pltpu.SemaphoreType.DMA,
            pltpu.SemaphoreType.DMA,
        ],
    )


def _gather(rowr, colr, taba, tabb):
    return _gather_kernel()(rowr, colr, taba, tabb)


EB = 1280


def _edge_body(u_ref, rel_ref, ea_ref, wd_ref, we_ref,
               b1_ref, we2_ref, b2_ref, m_ref):
    rel = rel_ref[...]
    d2 = jnp.sum(rel * rel, axis=1, keepdims=True)
    pre = (u_ref[...] + d2 * wd_ref[...]
           + jnp.dot(ea_ref[...], we_ref[...], preferred_element_type=_f32)
           + b1_ref[...])
    m1 = pre * jax.nn.sigmoid(pre)
    z = jnp.dot(m1, we2_ref[...], preferred_element_type=_f32) + b2_ref[...]
    m_ref[...] = z * jax.nn.sigmoid(z)


def _edge_mlp(u, rel, ea, wd, we, b1, we2, b2):
    full = lambda s: pl.BlockSpec(s, lambda i: (0, 0))
    return pl.pallas_call(
        _edge_body,
        grid=(E // EB,),
        in_specs=[
            pl.BlockSpec((EB, D), lambda i: (i, 0)),
            pl.BlockSpec((EB, CW), lambda i: (i, 0)),
            pl.BlockSpec((EB, D_EDGE), lambda i: (i, 0)),
            full((1, D)), full((D_EDGE, D)), full((1, D)),
            full((D, D)), full((1, D)),
        ],
        out_specs=pl.BlockSpec((EB, D), lambda i: (i, 0)),
        out_shape=jax.ShapeDtypeStruct((E, D), _f32),
    )(u, rel, ea, wd, we, b1, we2, b2)


HALF = N // NC
HALFP = HALF + ZR
EPT = E // NS
CHD = 80
NCHD = EPT // CHD
NZCH_A = HALFP // ZR
NZCH_D = HALF // ZR
ZT_A = (NZCH_A + NS - 1) // NS
ZT_D = (NZCH_D + NS - 1) // NS


def _scatter_body(rowd_hbm, m_hbm, outp_hbm, idxq, mbuf0, mbuf1, zbuf,
                  aggs, sem0, sem1):
    c = lax.axis_index("c")
    s = lax.axis_index("s")
    mbuf = (mbuf0, mbuf1)
    sem = (sem0, sem1)

    pltpu.sync_copy(rowd_hbm.at[s], idxq)

    lo = c * HALF

    def route(r, carry):
        for k in range(CHD // L):
            v = idxq[r, pl.ds(k * L, L)] - lo
            ok = (v >= 0) & (v < HALF)
            idxq[r, pl.ds(k * L, L)] = jnp.where(ok, v, HALF)
        return carry

    lax.fori_loop(0, NCHD, route, 0)

    def zb(i, carry):
        for k in range(D // L):
            zbuf[i, pl.ds(k * L, L)] = jnp.zeros((L,), _f32)
        return carry

    lax.fori_loop(0, ZR, zb, 0)
    for t in range(ZT_A):
        q = s + NS * t

        @pl.when(q < NZCH_A)
        def _():
            pltpu.sync_copy(zbuf, aggs.at[pl.ds(q * ZR, ZR)])
    plsc.subcore_barrier()

    def fire(j, st):
        pltpu.async_copy(m_hbm.at[pl.ds(s * EPT + j * CHD, CHD)],
                         mbuf[st], sem[st])

    def wait_l(st):
        pltpu.make_async_copy(m_hbm.at[pl.ds(s * EPT, CHD)],
                              mbuf[st], sem[st]).wait()

    last = NCHD - 1
    fire(0, 0)

    def body(i, carry):
        j = 2 * i
        wait_l(0)
        fire(j + 1, 1)
        pltpu.sync_copy(mbuf[0], aggs.at[idxq.at[j]], add=True)
        j2 = j + 1
        wait_l(1)
        fire(jnp.minimum(j2 + 1, last), 0)
        pltpu.sync_copy(mbuf[1], aggs.at[idxq.at[j2]], add=True)
        return carry

    lax.fori_loop(0, NCHD // 2, body, 0)
    wait_l(0)
    plsc.subcore_barrier()

    for t in range(ZT_D):
        q = s + NS * t

        @pl.when(q < NZCH_D)
        def _():
            pltpu.sync_copy(aggs.at[pl.ds(q * ZR, ZR)], zbuf)
            pltpu.sync_copy(zbuf, outp_hbm.at[c, pl.ds(q * ZR, ZR)])


@functools.lru_cache(maxsize=None)
def _scatter_kernel():
    return pl.kernel(
        _scatter_body,
        out_type=jax.ShapeDtypeStruct((NC, HALF, D), _f32),
        mesh=_sc_mesh(),
        scratch_types=[
            pltpu.VMEM((NCHD, CHD), jnp.int32),
            pltpu.VMEM((CHD, D), _f32),
            pltpu.VMEM((CHD, D), _f32),
            pltpu.VMEM((ZR, D), _f32),
            pltpu.VMEM_SHARED((HALFP, D), _f32),
            pltpu.SemaphoreType.DMA,
            pltpu.SemaphoreType.DMA,
        ],
    )


def _scatter(rowd, m):
    return _scatter_kernel()(rowd, m)


NBE = 2000


def _node_body(h_ref, a_ref, b_ref, wn1a_ref, wn1b_ref, bn1_ref,
               wn2_ref, bn2_ref, wh1_ref, bh1_ref, wh2_ref, bh2_ref,
               out_ref, acc_ref, cnt_ref):
    i = pl.program_id(0)
    t = (jnp.dot(h_ref[...], wn1a_ref[...], preferred_element_type=_f32)
         + jnp.dot(a_ref[...], wn1b_ref[...], preferred_element_type=_f32)
         + bn1_ref[...])
    t = t * jax.nn.sigmoid(t)
    hn = h_ref[...] + jnp.dot(t, wn2_ref[...],
                              preferred_element_type=_f32) + bn2_ref[...]
    oh = (b_ref[...] == lax.broadcasted_iota(jnp.int32, (1, G), 1))
    oh = oh.astype(_f32)
    p = lax.dot_general(oh, hn, (((0,), (0,)), ((), ())))
    ones = jnp.ones((NBE, 1), _f32)
    pc = lax.dot_general(oh, ones, (((0,), (0,)), ((), ())))

    @pl.when(i == 0)
    def _():
        acc_ref[...] = p
        cnt_ref[...] = pc

    @pl.when(i > 0)
    def _():
        acc_ref[...] += p
        cnt_ref[...] += pc

    @pl.when(i == pl.num_programs(0) - 1)
    def _():
        gh = acc_ref[...] / jnp.maximum(cnt_ref[...], 1.0)
        t2 = jnp.dot(gh, wh1_ref[...],
                     preferred_element_type=_f32) + bh1_ref[...]
        t2 = t2 * jax.nn.sigmoid(t2)
        out_ref[...] = jnp.dot(t2, wh2_ref[...],
                               preferred_element_type=_f32) + bh2_ref[...]


def _node_pool_head(h, agg, batch2d, wn1a, wn1b, bn1, wn2, bn2,
                    wh1, bh1, wh2, bh2):
    full = lambda s: pl.BlockSpec(s, lambda i: (0, 0))
    return pl.pallas_call(
        _node_body,
        grid=(N // NBE,),
        in_specs=[
            pl.BlockSpec((NBE, D), lambda i: (i, 0)),
            pl.BlockSpec((NBE, D), lambda i: (i, 0)),
            pl.BlockSpec((NBE, 1), lambda i: (i, 0)),
            full((D, D)), full((D, D)), full((1, D)),
            full((D, D)), full((1, D)),
            full((D, D)), full((1, D)), full((D, 1)), full((1, 1)),
        ],
        out_specs=pl.BlockSpec((G, 1), lambda i: (0, 0)),
        out_shape=jax.ShapeDtypeStruct((G, 1), _f32),
        scratch_shapes=[pltpu.VMEM((G, D), _f32), pltpu.VMEM((G, 1), _f32)],
    )(h, agg, batch2d, wn1a, wn1b, bn1, wn2, bn2, wh1, bh1, wh2, bh2)


def kernel(atom_feats, coord, edge_index, edge_attr, batch,
           W_emb, b_emb, W_e1, b_e1, W_e2, b_e2, W_c1, b_c1, W_c2,
           W_n1, b_n1, W_n2, b_n2, W_h1, b_h1, W_h2, b_h2):
    del W_c1, b_c1, W_c2
    rowr = edge_index[0].reshape(NW, NCHUNK, CH)
    colr = edge_index[1].reshape(NW, NCHUNK, CH)
    rowd = edge_index[0].reshape(NS, NCHD, CHD)
    cpad = jnp.pad(coord, ((0, 0), (0, D - 3)))

    wa = W_e1[:D]
    wb = W_e1[D:2 * D]
    wd = W_e1[2 * D:2 * D + 1]
    we = W_e1[2 * D + 1:]

    h, ta, tb = _nodeproj(atom_feats, cpad, W_emb, b_emb.reshape(1, D),
                          wa, wb)
    u, rel1d = _gather(rowr, colr, ta, tb)
    m = _edge_mlp(u, rel1d.reshape(E, CW), edge_attr,
                  wd, we, b_e1.reshape(1, D), W_e2, b_e2.reshape(1, D))
    aggp = _scatter(rowd, m)
    out = _node_pool_head(
        h, aggp.reshape(N, D), batch.reshape(N, 1),
        W_n1[:D], W_n1[D:], b_n1.reshape(1, D), W_n2, b_n2.reshape(1, D),
        W_h1, b_h1.reshape(1, D), W_h2, b_h2.reshape(1, 1))
    return out


# speedup vs baseline: 1.1645x; 1.1645x over previous
import functools

import jax
import jax.numpy as jnp
from jax import lax
from jax.experimental import pallas as pl
from jax.experimental.pallas import tpu as pltpu
from jax.experimental.pallas import tpu_sc as plsc

N = 10000
E = 320000
D_IN = 32
D = 128
D_EDGE = 16
G = 64

NC = 2
NS = 16
NW = NC * NS
L = 16

EPW = E // NW
CH = 40
NCHUNK = EPW // CH
ZR = 200
NZCH = N // ZR
ZT = (NZCH + NS - 1) // NS

_f32 = jnp.float32


@functools.lru_cache(maxsize=None)
def _sc_mesh():
    return plsc.VectorSubcoreMesh(
        core_axis_name="c", subcore_axis_name="s",
        num_cores=NC, num_subcores=NS)


NBA = 2000


TW = 2 * D


def _nodeproj_body(x_ref, cpad_ref, wemb_ref, bemb_ref, wa_ref, wb_ref,
                   h_ref, ta_ref, tb_ref):
    h = jnp.dot(x_ref[...], wemb_ref[...],
                preferred_element_type=_f32) + bemb_ref[...]
    h_ref[...] = h
    cp = cpad_ref[...]
    ha = jnp.dot(h, wa_ref[...], preferred_element_type=_f32)
    hb = jnp.dot(h, wb_ref[...], preferred_element_type=_f32)
    ta_ref[...] = jnp.concatenate([ha, cp], axis=1)
    tb_ref[...] = jnp.concatenate([hb, cp], axis=1)


def _nodeproj(x, cpad, wemb, bemb, wa, wb):
    full = lambda s: pl.BlockSpec(s, lambda i: (0, 0))
    return pl.pallas_call(
        _nodeproj_body,
        grid=(N // NBA,),
        in_specs=[
            pl.BlockSpec((NBA, D_IN), lambda i: (i, 0)),
            pl.BlockSpec((NBA, D), lambda i: (i, 0)),
            full((D_IN, D)), full((1, D)), full((D, D)), full((D, D)),
        ],
        out_specs=[
            pl.BlockSpec((NBA, D), lambda i: (i, 0)),
            pl.BlockSpec((NBA, TW), lambda i: (i, 0)),
            pl.BlockSpec((NBA, TW), lambda i: (i, 0)),
        ],
        out_shape=[
            jax.ShapeDtypeStruct((N, D), _f32),
            jax.ShapeDtypeStruct((N, TW), _f32),
            jax.ShapeDtypeStruct((N, TW), _f32),
        ],
    )(x, cpad, wemb, bemb, wa, wb)


CW = 16
_u8 = jnp.uint8


def _gather_body(rowr_hbm, colr_hbm, taba_hbm, tabb_hbm,
                 u_hbm, rel_hbm,
                 idxr, idxc, bufa0, bufb0, bufa1, bufb1, ubuf, rbuf,
                 sema0, semb0, sema1, semb1, semw):
    c = lax.axis_index("c")
    s = lax.axis_index("s")
    w = s * NC + c
    e_base = w * EPW

    pltpu.sync_copy(rowr_hbm.at[w], idxr)
    pltpu.sync_copy(colr_hbm.at[w], idxc)

    bufa = (bufa0, bufa1)
    bufb = (bufb0, bufb1)
    sema = (sema0, sema1)
    semb = (semb0, semb1)

    def fire(j, st):
        pltpu.async_copy(taba_hbm.at[idxr.at[j]], bufa[st], sema[st])
        pltpu.async_copy(tabb_hbm.at[idxc.at[j]], bufb[st], semb[st])

    def wait_g(st):
        pltpu.make_async_copy(taba_hbm.at[idxr.at[0]], bufa[st],
                              sema[st]).wait()
        pltpu.make_async_copy(tabb_hbm.at[idxc.at[0]], bufb[st],
                              semb[st]).wait()

    def compute(st):
        a = bufa[st]
        b = bufb[st]

        def crow(r, carry):
            for k in range(D // L):
                ubuf[r, pl.ds(k * L, L)] = (a[r, pl.ds(k * L, L)]
                                            + b[r, pl.ds(k * L, L)])
            rbuf[r, pl.ds(0, L)] = (a[r, pl.ds(D, L)]
                                    - b[r, pl.ds(D, L)])
            return carry

        lax.fori_loop(0, CH, crow, 0)

    def fire_w(j):
        e0 = e_base + j * CH
        pltpu.async_copy(ubuf, u_hbm.at[pl.ds(e0, CH)], semw)
        pltpu.async_copy(rbuf, rel_hbm.at[pl.ds(e0, CH)], semw)

    def wait_w(j):
        e0 = e_base + j * CH
        pltpu.make_async_copy(ubuf, u_hbm.at[pl.ds(e0, CH)], semw).wait()
        pltpu.make_async_copy(rbuf, rel_hbm.at[pl.ds(e0, CH)], semw).wait()

    def zb(r, carry):
        for k in range(1, D // L):
            rbuf[r, pl.ds(k * L, L)] = jnp.zeros((L,), _f32)
        return carry

    lax.fori_loop(0, CH, zb, 0)

    fire(0, 0)
    wait_g(0)
    fire(1, 1)
    compute(0)
    fire_w(0)

    last = NCHUNK - 1

    def body(i, carry):
        j = 2 * i + 1
        wait_g(1)
        fire(j + 1, 0)
        wait_w(j - 1)
        compute(1)
        fire_w(j)
        j2 = j + 1
        wait_g(0)
        fire(jnp.minimum(j2 + 1, last), 1)
        wait_w(j2 - 1)
        compute(0)
        fire_w(j2)
        return carry

    lax.fori_loop(0, (NCHUNK - 2) // 2, body, 0)
    wait_g(1)
    wait_w(last - 1)
    compute(1)
    fire_w(last)
    wait_w(last)


@functools.lru_cache(maxsize=None)
def _gather_kernel():
    return pl.kernel(
        _gather_body,
        out_type=[
            jax.ShapeDtypeStruct((E, D), _f32),
            jax.ShapeDtypeStruct((E, D), _f32),
        ],
        mesh=_sc_mesh(),
        scratch_types=[
            pltpu.VMEM((NCHUNK, CH), jnp.int32),
            pltpu.VMEM((NCHUNK, CH), jnp.int32),
            pltpu.VMEM((CH, TW), _f32),
            pltpu.VMEM((CH, TW), _f32),
            pltpu.VMEM((CH, TW), _f32),
            pltpu.VMEM((CH, TW), _f32),
            pltpu.VMEM((CH, D), _f32),
            pltpu.VMEM((CH, D), _f32),
            pltpu.SemaphoreType.DMA,
            pltpu.SemaphoreType.DMA,
            pltpu.SemaphoreType.DMA,
            pltpu.SemaphoreType.DMA,
            pltpu.SemaphoreType.DMA,
        ],
    )


def _gather(rowr, colr, taba, tabb):
    return _gather_kernel()(rowr, colr, taba, tabb)


EB = 1280


def _edge_body(u_ref, rel_ref, ea_ref, wd_ref, we_ref,
               b1_ref, we2_ref, b2_ref, m_ref):
    rel = rel_ref[...]
    d2 = jnp.sum(rel * rel, axis=1, keepdims=True)
    pre = (u_ref[...] + d2 * wd_ref[...]
           + jnp.dot(ea_ref[...], we_ref[...], preferred_element_type=_f32)
           + b1_ref[...])
    m1 = pre * jax.nn.sigmoid(pre)
    z = jnp.dot(m1, we2_ref[...], preferred_element_type=_f32) + b2_ref[...]
    m_ref[...] = z * jax.nn.sigmoid(z)


def _edge_mlp(u, rel, ea, wd, we, b1, we2, b2):
    full = lambda s: pl.BlockSpec(s, lambda i: (0, 0))
    return pl.pallas_call(
        _edge_body,
        grid=(E // EB,),
        in_specs=[
            pl.BlockSpec((EB, D), lambda i: (i, 0)),
            pl.BlockSpec((EB, D), lambda i: (i, 0)),
            pl.BlockSpec((EB, D_EDGE), lambda i: (i, 0)),
            full((1, D)), full((D_EDGE, D)), full((1, D)),
            full((D, D)), full((1, D)),
        ],
        out_specs=pl.BlockSpec((EB, D), lambda i: (i, 0)),
        out_shape=jax.ShapeDtypeStruct((E, D), _f32),
    )(u, rel, ea, wd, we, b1, we2, b2)


CHD = 80
NCHD = EPW // CHD
ZRD = 80
NZD = N // ZRD
ZTD = (NZD + NS - 1) // NS


def _scatter_body(rowd_hbm, m_hbm, outp_hbm, idxq, mbuf0, mbuf1,
                  aggs, sem0, sem1):
    c = lax.axis_index("c")
    s = lax.axis_index("s")
    w = s * NC + c
    mbuf = (mbuf0, mbuf1)
    sem = (sem0, sem1)

    pltpu.sync_copy(rowd_hbm.at[w], idxq)

    def zb(i, carry):
        for k in range(D // L):
            mbuf0[i, pl.ds(k * L, L)] = jnp.zeros((L,), _f32)
        return carry

    lax.fori_loop(0, ZRD, zb, 0)
    for t in range(ZTD):
        q = s + NS * t

        @pl.when(q < NZD)
        def _():
            pltpu.sync_copy(mbuf0, aggs.at[pl.ds(q * ZRD, ZRD)])
    plsc.subcore_barrier()

    def fire(j, st):
        pltpu.async_copy(m_hbm.at[pl.ds(w * EPW + j * CHD, CHD)],
                         mbuf[st], sem[st])

    def wait_l(st):
        pltpu.make_async_copy(m_hbm.at[pl.ds(w * EPW, CHD)],
                              mbuf[st], sem[st]).wait()

    last = NCHD - 1

    fire(0, 0)
    wait_l(0)
    fire(1, 1)
    pltpu.sync_copy(mbuf[0], aggs.at[idxq.at[0]], add=True)

    def body(i, carry):
        j = 2 * i + 1
        wait_l(1)
        fire(j + 1, 0)
        pltpu.sync_copy(mbuf[1], aggs.at[idxq.at[j]], add=True)
        j2 = j + 1
        wait_l(0)
        fire(jnp.minimum(j2 + 1, last), 1)
        pltpu.sync_copy(mbuf[0], aggs.at[idxq.at[j2]], add=True)
        return carry

    lax.fori_loop(0, (NCHD - 1) // 2, body, 0)
    wait_l(1)
    plsc.subcore_barrier()

    for t in range(ZTD):
        q = s + NS * t

        @pl.when(q < NZD)
        def _():
            pltpu.sync_copy(aggs.at[pl.ds(q * ZRD, ZRD)], mbuf0)
            pltpu.sync_copy(mbuf0, outp_hbm.at[c, pl.ds(q * ZRD, ZRD)])


@functools.lru_cache(maxsize=None)
def _scatter_kernel():
    return pl.kernel(
        _scatter_body,
        out_type=jax.ShapeDtypeStruct((NC, N, D), _f32),
        mesh=_sc_mesh(),
        scratch_types=[
            pltpu.VMEM((NCHD, CHD), jnp.int32),
            pltpu.VMEM((CHD, D), _f32),
            pltpu.VMEM((CHD, D), _f32),
            pltpu.VMEM_SHARED((N, D), _f32),
            pltpu.SemaphoreType.DMA,
            pltpu.SemaphoreType.DMA,
        ],
    )


def _scatter(rowd, m):
    return _scatter_kernel()(rowd, m)


NBE = 2000


def _node_body(h_ref, a0_ref, a1_ref, b_ref, wn1a_ref, wn1b_ref, bn1_ref,
               wn2_ref, bn2_ref, wh1_ref, bh1_ref, wh2_ref, bh2_ref,
               out_ref, acc_ref, cnt_ref):
    i = pl.program_id(0)
    agg = a0_ref[...] + a1_ref[...]
    t = (jnp.dot(h_ref[...], wn1a_ref[...], preferred_element_type=_f32)
         + jnp.dot(agg, wn1b_ref[...], preferred_element_type=_f32)
         + bn1_ref[...])
    t = t * jax.nn.sigmoid(t)
    hn = h_ref[...] + jnp.dot(t, wn2_ref[...],
                              preferred_element_type=_f32) + bn2_ref[...]
    oh = (b_ref[...] == lax.broadcasted_iota(jnp.int32, (1, G), 1))
    oh = oh.astype(_f32)
    p = lax.dot_general(oh, hn, (((0,), (0,)), ((), ())))
    ones = jnp.ones((NBE, 1), _f32)
    pc = lax.dot_general(oh, ones, (((0,), (0,)), ((), ())))

    @pl.when(i == 0)
    def _():
        acc_ref[...] = p
        cnt_ref[...] = pc

    @pl.when(i > 0)
    def _():
        acc_ref[...] += p
        cnt_ref[...] += pc

    @pl.when(i == pl.num_programs(0) - 1)
    def _():
        gh = acc_ref[...] / jnp.maximum(cnt_ref[...], 1.0)
        t2 = jnp.dot(gh, wh1_ref[...],
                     preferred_element_type=_f32) + bh1_ref[...]
        t2 = t2 * jax.nn.sigmoid(t2)
        out_ref[...] = jnp.dot(t2, wh2_ref[...],
                               preferred_element_type=_f32) + bh2_ref[...]


def _node_pool_head(h, a0, a1, batch2d, wn1a, wn1b, bn1, wn2, bn2,
                    wh1, bh1, wh2, bh2):
    full = lambda s: pl.BlockSpec(s, lambda i: (0, 0))
    return pl.pallas_call(
        _node_body,
        grid=(N // NBE,),
        in_specs=[
            pl.BlockSpec((NBE, D), lambda i: (i, 0)),
            pl.BlockSpec((NBE, D), lambda i: (i, 0)),
            pl.BlockSpec((NBE, D), lambda i: (i, 0)),
            pl.BlockSpec((NBE, 1), lambda i: (i, 0)),
            full((D, D)), full((D, D)), full((1, D)),
            full((D, D)), full((1, D)),
            full((D, D)), full((1, D)), full((D, 1)), full((1, 1)),
        ],
        out_specs=pl.BlockSpec((G, 1), lambda i: (0, 0)),
        out_shape=jax.ShapeDtypeStruct((G, 1), _f32),
        scratch_shapes=[pltpu.VMEM((G, D), _f32), pltpu.VMEM((G, 1), _f32)],
    )(h, a0, a1, batch2d, wn1a, wn1b, bn1, wn2, bn2, wh1, bh1, wh2, bh2)


def kernel(atom_feats, coord, edge_index, edge_attr, batch,
           W_emb, b_emb, W_e1, b_e1, W_e2, b_e2, W_c1, b_c1, W_c2,
           W_n1, b_n1, W_n2, b_n2, W_h1, b_h1, W_h2, b_h2):
    del W_c1, b_c1, W_c2
    rowr = edge_index[0].reshape(NW, NCHUNK, CH)
    colr = edge_index[1].reshape(NW, NCHUNK, CH)
    rowd = edge_index[0].reshape(NW, NCHD, CHD)
    cpad = jnp.pad(coord, ((0, 0), (0, D - 3)))

    wa = W_e1[:D]
    wb = W_e1[D:2 * D]
    wd = W_e1[2 * D:2 * D + 1]
    we = W_e1[2 * D + 1:]

    h, ta, tb = _nodeproj(atom_feats, cpad, W_emb, b_emb.reshape(1, D),
                          wa, wb)
    u, rel = _gather(rowr, colr, ta, tb)
    m = _edge_mlp(u, rel, edge_attr,
                  wd, we, b_e1.reshape(1, D), W_e2, b_e2.reshape(1, D))
    aggp = _scatter(rowd, m)
    out = _node_pool_head(
        h, aggp[0], aggp[1], batch.reshape(N, 1),
        W_n1[:D], W_n1[D:], b_n1.reshape(1, D), W_n2, b_n2.reshape(1, D),
        W_h1, b_h1.reshape(1, D), W_h2, b_h2.reshape(1, 1))
    return out
